# baseline (speedup 1.0000x reference)
import functools

import jax
import jax.numpy as jnp
from jax import lax
from jax.experimental import pallas as pl
from jax.experimental.pallas import tpu as pltpu
from jax.experimental.pallas import tpu_sc as plsc

H = 12
K = 64
ROW_TILE = 512
NEGF = -1e30


def _head_onehot(C):
    dh = C // H
    cidx = lax.broadcasted_iota(jnp.int32, (H, C), 1) // dh
    hidx = lax.broadcasted_iota(jnp.int32, (H, C), 0)
    return (cidx == hidx).astype(jnp.float32)


def _body_a(x_ref, wq_ref, bq_ref, wkv_ref, bkv_ref, scores_ref):
    N, C = x_ref.shape[1], x_ref.shape[2]
    T = wq_ref.shape[0]
    dh = C // H
    scale = dh ** (-0.5)
    f32 = jnp.float32

    X = x_ref[0]
    kmat = lax.dot_general(X, wkv_ref[0:C, :], (((1,), (1,)), ((), ())),
                           preferred_element_type=f32) + bkv_ref[0]

    qs = []
    for t in range(T):
        qt = lax.dot_general(X[t:t + 1, :], wq_ref[t],
                             (((1,), (1,)), ((), ())),
                             preferred_element_type=f32)
        qs.append(qt + bq_ref[t])
    q = jnp.concatenate(qs, axis=0)

    hmask = _head_onehot(C)
    Qexp = (q[:, None, :] * hmask[None, :, :]).reshape(T * H, C)
    scores = lax.dot_general(Qexp, kmat, (((1,), (1,)), ((), ())),
                             preferred_element_type=f32) * scale
    col = lax.broadcasted_iota(jnp.int32, (T * H, N), 1)
    scores_ref[0] = jnp.where(col >= T, scores, f32(NEGF))


def _sc_body(scores_hbm, w_hbm, row_v, us_v, wrow_v, hist_v):
    R, N = scores_hbm.shape
    NC = 2
    wid = lax.axis_index("s") * NC + lax.axis_index("c")
    nw = 32
    U = 8
    nchunk = N // 16
    lane = lax.iota(jnp.int32, 16)
    ones16 = jnp.ones((16,), jnp.int32)
    zeros16 = jnp.zeros((16,), jnp.int32)
    rows_per = (R + nw - 1) // nw

    def do_row(i, _):
        r = wid + nw * i

        @pl.when(r < R)
        def _():
            pltpu.sync_copy(scores_hbm.at[r], row_v)

            def p0(j, mvec):
                for j2 in range(U):
                    o = (j * U + j2) * 16
                    v = row_v[pl.ds(o, 16)]
                    b = lax.bitcast_convert_type(v, jnp.int32)
                    us = jnp.where(b >= 0, b, b ^ jnp.int32(0x7FFFFFFF))
                    us_v[pl.ds(o, 16)] = us
                    mvec = jnp.maximum(mvec, v)
                return mvec

            mvec = lax.fori_loop(0, nchunk // U, p0,
                                 jnp.full((16,), jnp.float32(NEGF)))
            m = jnp.max(mvec)

            prefix = jnp.int32(0)
            remaining = jnp.int32(K)
            for l in range(8):
                sh = 28 - 4 * l
                for b in range(16):
                    hist_v[pl.ds(b * 16, 16)] = zeros16

                if l == 0:
                    def scan(j, _):
                        for j2 in range(U):
                            o = (j * U + j2) * 16
                            us = us_v[pl.ds(o, 16)]
                            bucket = (jnp.right_shift(us, jnp.int32(sh))
                                      & jnp.int32(15)) ^ jnp.int32(8)
                            plsc.addupdate_scatter(
                                hist_v, [bucket * 16 + lane], ones16)
                        return 0

                    lax.fori_loop(0, nchunk // U, scan, 0)
                else:
                    hm = jnp.int32(-1 << (sh + 4))

                    def scan(j, _, _prefix=prefix, _hm=hm, _sh=sh):
                        for j2 in range(U):
                            o = (j * U + j2) * 16
                            us = us_v[pl.ds(o, 16)]
                            member = ((us ^ _prefix) & _hm) == 0
                            bucket = (jnp.right_shift(us, jnp.int32(_sh))
                                      & jnp.int32(15))
                            plsc.addupdate_scatter(
                                hist_v, [bucket * 16 + lane], ones16,
                                mask=member)
                        return 0

                    lax.fori_loop(0, nchunk // U, scan, 0)

                selb = jnp.int32(0)
                found = jnp.int32(0)
                for b in range(15, -1, -1):
                    tot = jnp.sum(hist_v[pl.ds(b * 16, 16)])
                    take = jnp.logical_and(found == 0, tot >= remaining)
                    selb = jnp.where(take, jnp.int32(b), selb)
                    remaining = jnp.where(
                        jnp.logical_or(found == 1, take),
                        remaining, remaining - tot)
                    found = jnp.where(take, jnp.int32(1), found)
                raw_b = selb ^ jnp.int32(8) if l == 0 else selb
                prefix = prefix | jnp.left_shift(raw_b, jnp.int32(sh))

            theta = prefix

            def pw(j, zvec):
                for j2 in range(U):
                    o = (j * U + j2) * 16
                    v = row_v[pl.ds(o, 16)]
                    us = us_v[pl.ds(o, 16)]
                    e = jnp.exp(v - m)
                    w = jnp.where(us >= theta, e, jnp.float32(0.0))
                    wrow_v[pl.ds(o, 16)] = w
                    zvec = zvec + w
                return zvec

            zvec = lax.fori_loop(0, nchunk // U, pw,
                                 jnp.zeros((16,), jnp.float32))
            invz = jnp.ones((16,), jnp.float32) / jnp.full(
                (16,), jnp.sum(zvec), jnp.float32)

            def pn(j, _):
                for j2 in range(U):
                    o = (j * U + j2) * 16
                    wrow_v[pl.ds(o, 16)] = wrow_v[pl.ds(o, 16)] * invz
                return 0

            lax.fori_loop(0, nchunk // U, pn, 0)
            pltpu.sync_copy(wrow_v, w_hbm.at[r])

        return 0

    lax.fori_loop(0, rows_per, do_row, 0)


def _body_b(x_ref, wf_ref, wkv_ref, bkv_ref, we_ref, out_ref):
    N, C = x_ref.shape[0], x_ref.shape[1]
    f32 = jnp.float32
    t = pl.program_id(0)

    X = x_ref[...]
    wfield = wf_ref[0]
    hmask = _head_onehot(C)

    u = lax.dot_general(wfield, X, (((1,), (0,)), ((), ())),
                        preferred_element_type=f32)
    uv = lax.dot_general(u, wkv_ref[C:2 * C, :], (((1,), (1,)), ((), ())),
                         preferred_element_type=f32)
    attn = jnp.sum(uv * hmask, axis=0, keepdims=True) + bkv_ref[1]
    tok = lax.dot_general(attn, we_ref[0], (((1,), (1,)), ((), ())),
                          preferred_element_type=f32)

    first = t == 0
    for r0 in range(0, N, ROW_TILE):
        wslice = wfield[:, r0:r0 + ROW_TILE]
        Wf = lax.dot_general(wslice, hmask, (((0,), (0,)), ((), ())),
                             preferred_element_type=f32)
        Yc = lax.dot_general(X[r0:r0 + ROW_TILE, :] * Wf, we_ref[0],
                             (((1,), (1,)), ((), ())),
                             preferred_element_type=f32)

        @pl.when(first)
        def _():
            out_ref[r0:r0 + ROW_TILE, :] = Yc

        @pl.when(jnp.logical_not(first))
        def _():
            out_ref[r0:r0 + ROW_TILE, :] += Yc

    out_ref[pl.ds(t, 1), :] = tok


@jax.jit
def kernel(x, Wq, bq, Wkv, bkv, We):
    B, N, C = x.shape
    T = Wq.shape[0]
    f32 = jnp.float32
    R = T * H

    bq3 = bq.reshape(T, 1, C)
    bkv3 = bkv.reshape(2, 1, C)

    call_a = pl.pallas_call(
        _body_a,
        grid=(1,),
        in_specs=[
            pl.BlockSpec((1, N, C), lambda b: (b, 0, 0)),
            pl.BlockSpec((T, C, C), lambda b: (0, 0, 0)),
            pl.BlockSpec((T, 1, C), lambda b: (0, 0, 0)),
            pl.BlockSpec((2 * C, C), lambda b: (0, 0)),
            pl.BlockSpec((2, 1, C), lambda b: (0, 0, 0)),
        ],
        out_specs=pl.BlockSpec((1, T * H, N), lambda b: (b, 0, 0)),
        out_shape=jax.ShapeDtypeStruct((1, T * H, N), f32),
        compiler_params=pltpu.CompilerParams(
            vmem_limit_bytes=63 * 1024 * 1024),
    )

    sc_topk = functools.partial(
        pl.kernel,
        out_type=jax.ShapeDtypeStruct((R, N), f32),
        mesh=plsc.VectorSubcoreMesh(core_axis_name="c", subcore_axis_name="s"),
        compiler_params=pltpu.CompilerParams(needs_layout_passes=False),
        scratch_types=[
            pltpu.VMEM((N,), f32),
            pltpu.VMEM((N,), jnp.int32),
            pltpu.VMEM((N,), f32),
            pltpu.VMEM((256,), jnp.int32),
        ],
    )(_sc_body)

    call_b = pl.pallas_call(
        _body_b,
        grid=(T,),
        in_specs=[
            pl.BlockSpec((N, C), lambda t: (0, 0)),
            pl.BlockSpec((1, H, N), lambda t: (t, 0, 0)),
            pl.BlockSpec((2 * C, C), lambda t: (0, 0)),
            pl.BlockSpec((2, 1, C), lambda t: (0, 0, 0)),
            pl.BlockSpec((1, C, C), lambda t: (t, 0, 0)),
        ],
        out_specs=pl.BlockSpec((N, C), lambda t: (0, 0)),
        out_shape=jax.ShapeDtypeStruct((N, C), x.dtype),
        compiler_params=pltpu.CompilerParams(
            vmem_limit_bytes=63 * 1024 * 1024),
    )

    wfs = []
    for b in range(B):
        scores_b = call_a(x[b:b + 1], Wq, bq3, Wkv, bkv3)
        wfs.append(sc_topk(scores_b.reshape(R, N)))
    outs = []
    for b in range(B):
        outs.append(call_b(x[b], wfs[b].reshape(T, H, N), Wkv, bkv3, We))
    return jnp.stack(outs, axis=0)


# speedup vs baseline: 1.2682x; 1.2682x over previous
import functools

import jax
import jax.numpy as jnp
from jax import lax
from jax.experimental import pallas as pl
from jax.experimental.pallas import tpu as pltpu
from jax.experimental.pallas import tpu_sc as plsc

H = 12
K = 64
ROW_TILE = 1024
NEGF = -1e30


def _head_onehot(C):
    dh = C // H
    cidx = lax.broadcasted_iota(jnp.int32, (H, C), 1) // dh
    hidx = lax.broadcasted_iota(jnp.int32, (H, C), 0)
    return (cidx == hidx).astype(jnp.float32)


def _body_a(x_ref, wq_ref, bq_ref, wkv_ref, bkv_ref, scores_ref):
    N, C = x_ref.shape[1], x_ref.shape[2]
    T = wq_ref.shape[0]
    dh = C // H
    scale = dh ** (-0.5)
    f32 = jnp.float32

    X = x_ref[0]
    kmat = lax.dot_general(X, wkv_ref[0:C, :], (((1,), (1,)), ((), ())),
                           preferred_element_type=f32) + bkv_ref[0]

    qs = []
    for t in range(T):
        qt = lax.dot_general(X[t:t + 1, :], wq_ref[t],
                             (((1,), (1,)), ((), ())),
                             preferred_element_type=f32)
        qs.append(qt + bq_ref[t])
    q = jnp.concatenate(qs, axis=0)

    hmask = _head_onehot(C)
    Qexp = (q[:, None, :] * hmask[None, :, :]).reshape(T * H, C)
    scores = lax.dot_general(Qexp, kmat, (((1,), (1,)), ((), ())),
                             preferred_element_type=f32) * scale
    col = lax.broadcasted_iota(jnp.int32, (T * H, N), 1)
    scores_ref[0] = jnp.where(col >= T, scores, f32(NEGF))


def _sc_body(scores_hbm, w_hbm, row_v, us_v, wrow_v, hist_v):
    R, N = scores_hbm.shape
    NC = 2
    wid = lax.axis_index("s") * NC + lax.axis_index("c")
    nw = 32
    U = 8
    nchunk = N // 16
    lane = lax.iota(jnp.int32, 16)
    ones16 = jnp.ones((16,), jnp.int32)
    zeros16 = jnp.zeros((16,), jnp.int32)
    rows_per = (R + nw - 1) // nw

    def do_row(i, _):
        r = wid + nw * i

        @pl.when(r < R)
        def _():
            pltpu.sync_copy(scores_hbm.at[r], row_v)

            def p0(j, mvec):
                for j2 in range(U):
                    o = (j * U + j2) * 16
                    v = row_v[pl.ds(o, 16)]
                    b = lax.bitcast_convert_type(v, jnp.int32)
                    us = jnp.where(b >= 0, b, b ^ jnp.int32(0x7FFFFFFF))
                    us_v[pl.ds(o, 16)] = us
                    mvec = jnp.maximum(mvec, v)
                return mvec

            mvec = lax.fori_loop(0, nchunk // U, p0,
                                 jnp.full((16,), jnp.float32(NEGF)))
            m = jnp.max(mvec)

            prefix = jnp.int32(0)
            remaining = jnp.int32(K)
            for l in range(8):
                sh = 28 - 4 * l
                for b in range(16):
                    hist_v[pl.ds(b * 16, 16)] = zeros16

                if l == 0:
                    def scan(j, _):
                        for j2 in range(U):
                            o = (j * U + j2) * 16
                            us = us_v[pl.ds(o, 16)]
                            bucket = (jnp.right_shift(us, jnp.int32(sh))
                                      & jnp.int32(15)) ^ jnp.int32(8)
                            plsc.addupdate_scatter(
                                hist_v, [bucket * 16 + lane], ones16)
                        return 0

                    lax.fori_loop(0, nchunk // U, scan, 0)
                else:
                    hm = jnp.int32(-1 << (sh + 4))

                    def scan(j, _, _prefix=prefix, _hm=hm, _sh=sh):
                        for j2 in range(U):
                            o = (j * U + j2) * 16
                            us = us_v[pl.ds(o, 16)]
                            member = ((us ^ _prefix) & _hm) == 0
                            bucket = (jnp.right_shift(us, jnp.int32(_sh))
                                      & jnp.int32(15))
                            plsc.addupdate_scatter(
                                hist_v, [bucket * 16 + lane], ones16,
                                mask=member)
                        return 0

                    lax.fori_loop(0, nchunk // U, scan, 0)

                selb = jnp.int32(0)
                found = jnp.int32(0)
                for b in range(15, -1, -1):
                    tot = jnp.sum(hist_v[pl.ds(b * 16, 16)])
                    take = jnp.logical_and(found == 0, tot >= remaining)
                    selb = jnp.where(take, jnp.int32(b), selb)
                    remaining = jnp.where(
                        jnp.logical_or(found == 1, take),
                        remaining, remaining - tot)
                    found = jnp.where(take, jnp.int32(1), found)
                raw_b = selb ^ jnp.int32(8) if l == 0 else selb
                prefix = prefix | jnp.left_shift(raw_b, jnp.int32(sh))

            theta = prefix

            def pw(j, zvec):
                for j2 in range(U):
                    o = (j * U + j2) * 16
                    v = row_v[pl.ds(o, 16)]
                    us = us_v[pl.ds(o, 16)]
                    e = jnp.exp(v - m)
                    w = jnp.where(us >= theta, e, jnp.float32(0.0))
                    wrow_v[pl.ds(o, 16)] = w
                    zvec = zvec + w
                return zvec

            zvec = lax.fori_loop(0, nchunk // U, pw,
                                 jnp.zeros((16,), jnp.float32))
            invz = jnp.ones((16,), jnp.float32) / jnp.full(
                (16,), jnp.sum(zvec), jnp.float32)

            def pn(j, _):
                for j2 in range(U):
                    o = (j * U + j2) * 16
                    wrow_v[pl.ds(o, 16)] = wrow_v[pl.ds(o, 16)] * invz
                return 0

            lax.fori_loop(0, nchunk // U, pn, 0)
            pltpu.sync_copy(wrow_v, w_hbm.at[r])

        return 0

    lax.fori_loop(0, rows_per, do_row, 0)


def _body_b(x_ref, wf_ref, wkv_ref, bkv_ref, we_ref, out_ref):
    N, C = x_ref.shape[1], x_ref.shape[2]
    f32 = jnp.float32
    t = pl.program_id(1)

    X = x_ref[0]
    wfield = wf_ref[0, 0]
    hmask = _head_onehot(C)

    u = lax.dot_general(wfield, X, (((1,), (0,)), ((), ())),
                        preferred_element_type=f32)
    uv = lax.dot_general(u, wkv_ref[C:2 * C, :], (((1,), (1,)), ((), ())),
                         preferred_element_type=f32)
    attn = jnp.sum(uv * hmask, axis=0, keepdims=True) + bkv_ref[1]
    tok = lax.dot_general(attn, we_ref[0], (((1,), (1,)), ((), ())),
                          preferred_element_type=f32)

    first = t == 0
    for r0 in range(0, N, ROW_TILE):
        wslice = wfield[:, r0:r0 + ROW_TILE]
        Wf = lax.dot_general(wslice, hmask, (((0,), (0,)), ((), ())),
                             preferred_element_type=f32)
        Yc = lax.dot_general(X[r0:r0 + ROW_TILE, :] * Wf, we_ref[0],
                             (((1,), (1,)), ((), ())),
                             preferred_element_type=f32)

        @pl.when(first)
        def _():
            out_ref[0, r0:r0 + ROW_TILE, :] = Yc

        @pl.when(jnp.logical_not(first))
        def _():
            out_ref[0, r0:r0 + ROW_TILE, :] += Yc

    out_ref[0, pl.ds(t, 1), :] = tok


@jax.jit
def kernel(x, Wq, bq, Wkv, bkv, We):
    B, N, C = x.shape
    T = Wq.shape[0]
    f32 = jnp.float32
    R = B * T * H

    scores = pl.pallas_call(
        _body_a,
        grid=(B,),
        in_specs=[
            pl.BlockSpec((1, N, C), lambda b: (b, 0, 0)),
            pl.BlockSpec((T, C, C), lambda b: (0, 0, 0)),
            pl.BlockSpec((T, 1, C), lambda b: (0, 0, 0)),
            pl.BlockSpec((2 * C, C), lambda b: (0, 0)),
            pl.BlockSpec((2, 1, C), lambda b: (0, 0, 0)),
        ],
        out_specs=pl.BlockSpec((1, T * H, N), lambda b: (b, 0, 0)),
        out_shape=jax.ShapeDtypeStruct((B, T * H, N), f32),
        compiler_params=pltpu.CompilerParams(
            vmem_limit_bytes=63 * 1024 * 1024),
    )(x, Wq, bq.reshape(T, 1, C), Wkv, bkv.reshape(2, 1, C))

    sc_topk = functools.partial(
        pl.kernel,
        out_type=jax.ShapeDtypeStruct((R, N), f32),
        mesh=plsc.VectorSubcoreMesh(core_axis_name="c", subcore_axis_name="s"),
        compiler_params=pltpu.CompilerParams(needs_layout_passes=False),
        scratch_types=[
            pltpu.VMEM((N,), f32),
            pltpu.VMEM((N,), jnp.int32),
            pltpu.VMEM((N,), f32),
            pltpu.VMEM((256,), jnp.int32),
        ],
    )(_sc_body)
    wfield = sc_topk(scores.reshape(R, N))

    out = pl.pallas_call(
        _body_b,
        grid=(B, T),
        in_specs=[
            pl.BlockSpec((1, N, C), lambda b, t: (b, 0, 0)),
            pl.BlockSpec((1, 1, H, N), lambda b, t: (b, t, 0, 0)),
            pl.BlockSpec((2 * C, C), lambda b, t: (0, 0)),
            pl.BlockSpec((2, 1, C), lambda b, t: (0, 0, 0)),
            pl.BlockSpec((1, C, C), lambda b, t: (t, 0, 0)),
        ],
        out_specs=pl.BlockSpec((1, N, C), lambda b, t: (b, 0, 0)),
        out_shape=jax.ShapeDtypeStruct((B, N, C), x.dtype),
        compiler_params=pltpu.CompilerParams(
            vmem_limit_bytes=63 * 1024 * 1024),
    )(x, wfield.reshape(B, T, H, N), Wkv, bkv.reshape(2, 1, C), We)
    return out


# speedup vs baseline: 1.2910x; 1.0180x over previous
import functools

import jax
import jax.numpy as jnp
from jax import lax
from jax.experimental import pallas as pl
from jax.experimental.pallas import tpu as pltpu
from jax.experimental.pallas import tpu_sc as plsc

H = 12
K = 64
ROW_TILE = 2048
NEGF = -1e30


def _head_onehot(C):
    dh = C // H
    cidx = lax.broadcasted_iota(jnp.int32, (H, C), 1) // dh
    hidx = lax.broadcasted_iota(jnp.int32, (H, C), 0)
    return (cidx == hidx).astype(jnp.float32)


def _body_a(x_ref, wq_ref, bq_ref, wkv_ref, bkv_ref, scores_ref):
    N, C = x_ref.shape[1], x_ref.shape[2]
    T = wq_ref.shape[0]
    dh = C // H
    scale = dh ** (-0.5)
    f32 = jnp.float32

    X = x_ref[0]
    kmat = lax.dot_general(X, wkv_ref[0:C, :], (((1,), (1,)), ((), ())),
                           preferred_element_type=f32) + bkv_ref[0]

    qs = []
    for t in range(T):
        qt = lax.dot_general(X[t:t + 1, :], wq_ref[t],
                             (((1,), (1,)), ((), ())),
                             preferred_element_type=f32)
        qs.append(qt + bq_ref[t])
    q = jnp.concatenate(qs, axis=0)

    hmask = _head_onehot(C)
    Qexp = (q[:, None, :] * hmask[None, :, :]).reshape(T * H, C)
    scores = lax.dot_general(Qexp, kmat, (((1,), (1,)), ((), ())),
                             preferred_element_type=f32) * scale
    col = lax.broadcasted_iota(jnp.int32, (T * H, N), 1)
    scores_ref[0] = jnp.where(col >= T, scores, f32(NEGF))


def _sc_body(scores_hbm, w_hbm, row_v, us_v, wrow_v, hist_v):
    R, N = scores_hbm.shape
    NC = 2
    wid = lax.axis_index("s") * NC + lax.axis_index("c")
    nw = 32
    U = 8
    nchunk = N // 16
    lane = lax.iota(jnp.int32, 16)
    ones16 = jnp.ones((16,), jnp.int32)
    zeros16 = jnp.zeros((16,), jnp.int32)
    rows_per = (R + nw - 1) // nw

    def do_row(i, _):
        r = wid + nw * i

        @pl.when(r < R)
        def _():
            pltpu.sync_copy(scores_hbm.at[r], row_v)

            def p0(j, mvec):
                for j2 in range(U):
                    o = (j * U + j2) * 16
                    v = row_v[pl.ds(o, 16)]
                    b = lax.bitcast_convert_type(v, jnp.int32)
                    us = jnp.where(b >= 0, b, b ^ jnp.int32(0x7FFFFFFF))
                    us_v[pl.ds(o, 16)] = us
                    mvec = jnp.maximum(mvec, v)
                return mvec

            mvec = lax.fori_loop(0, nchunk // U, p0,
                                 jnp.full((16,), jnp.float32(NEGF)))
            m = jnp.max(mvec)

            prefix = jnp.int32(0)
            remaining = jnp.int32(K)
            for l in range(8):
                sh = 28 - 4 * l
                for b in range(16):
                    hist_v[pl.ds(b * 16, 16)] = zeros16

                if l == 0:
                    def scan(j, _):
                        for j2 in range(U):
                            o = (j * U + j2) * 16
                            us = us_v[pl.ds(o, 16)]
                            bucket = (jnp.right_shift(us, jnp.int32(sh))
                                      & jnp.int32(15)) ^ jnp.int32(8)
                            plsc.addupdate_scatter(
                                hist_v, [bucket * 16 + lane], ones16)
                        return 0

                    lax.fori_loop(0, nchunk // U, scan, 0)
                else:
                    hm = jnp.int32(-1 << (sh + 4))

                    def scan(j, _, _prefix=prefix, _hm=hm, _sh=sh):
                        for j2 in range(U):
                            o = (j * U + j2) * 16
                            us = us_v[pl.ds(o, 16)]
                            member = ((us ^ _prefix) & _hm) == 0
                            bucket = (jnp.right_shift(us, jnp.int32(_sh))
                                      & jnp.int32(15))
                            plsc.addupdate_scatter(
                                hist_v, [bucket * 16 + lane], ones16,
                                mask=member)
                        return 0

                    lax.fori_loop(0, nchunk // U, scan, 0)

                selb = jnp.int32(0)
                found = jnp.int32(0)
                for b in range(15, -1, -1):
                    tot = jnp.sum(hist_v[pl.ds(b * 16, 16)])
                    take = jnp.logical_and(found == 0, tot >= remaining)
                    selb = jnp.where(take, jnp.int32(b), selb)
                    remaining = jnp.where(
                        jnp.logical_or(found == 1, take),
                        remaining, remaining - tot)
                    found = jnp.where(take, jnp.int32(1), found)
                raw_b = selb ^ jnp.int32(8) if l == 0 else selb
                prefix = prefix | jnp.left_shift(raw_b, jnp.int32(sh))

            theta = prefix

            def pw(j, zvec):
                for j2 in range(U):
                    o = (j * U + j2) * 16
                    v = row_v[pl.ds(o, 16)]
                    us = us_v[pl.ds(o, 16)]
                    e = jnp.exp(v - m)
                    w = jnp.where(us >= theta, e, jnp.float32(0.0))
                    wrow_v[pl.ds(o, 16)] = w
                    zvec = zvec + w
                return zvec

            zvec = lax.fori_loop(0, nchunk // U, pw,
                                 jnp.zeros((16,), jnp.float32))
            invz = jnp.ones((16,), jnp.float32) / jnp.full(
                (16,), jnp.sum(zvec), jnp.float32)

            def pn(j, _):
                for j2 in range(U):
                    o = (j * U + j2) * 16
                    wrow_v[pl.ds(o, 16)] = wrow_v[pl.ds(o, 16)] * invz
                return 0

            lax.fori_loop(0, nchunk // U, pn, 0)
            pltpu.sync_copy(wrow_v, w_hbm.at[r])

        return 0

    lax.fori_loop(0, rows_per, do_row, 0)


def _body_b(x_ref, wf_ref, wkv_ref, bkv_ref, we_ref, out_ref):
    N, C = x_ref.shape[1], x_ref.shape[2]
    f32 = jnp.float32
    t = pl.program_id(1)

    X = x_ref[0]
    wfield = wf_ref[0, 0]
    hmask = _head_onehot(C)

    u = lax.dot_general(wfield, X, (((1,), (0,)), ((), ())),
                        preferred_element_type=f32)
    uv = lax.dot_general(u, wkv_ref[C:2 * C, :], (((1,), (1,)), ((), ())),
                         preferred_element_type=f32)
    attn = jnp.sum(uv * hmask, axis=0, keepdims=True) + bkv_ref[1]
    tok = lax.dot_general(attn, we_ref[0], (((1,), (1,)), ((), ())),
                          preferred_element_type=f32)

    first = t == 0
    for r0 in range(0, N, ROW_TILE):
        wslice = wfield[:, r0:r0 + ROW_TILE]
        Wf = lax.dot_general(wslice, hmask, (((0,), (0,)), ((), ())),
                             preferred_element_type=f32)
        Yc = lax.dot_general(X[r0:r0 + ROW_TILE, :] * Wf, we_ref[0],
                             (((1,), (1,)), ((), ())),
                             preferred_element_type=f32)

        @pl.when(first)
        def _():
            out_ref[0, r0:r0 + ROW_TILE, :] = Yc

        @pl.when(jnp.logical_not(first))
        def _():
            out_ref[0, r0:r0 + ROW_TILE, :] += Yc

    out_ref[0, pl.ds(t, 1), :] = tok


@jax.jit
def kernel(x, Wq, bq, Wkv, bkv, We):
    B, N, C = x.shape
    T = Wq.shape[0]
    f32 = jnp.float32
    R = B * T * H

    scores = pl.pallas_call(
        _body_a,
        grid=(B,),
        in_specs=[
            pl.BlockSpec((1, N, C), lambda b: (b, 0, 0)),
            pl.BlockSpec((T, C, C), lambda b: (0, 0, 0)),
            pl.BlockSpec((T, 1, C), lambda b: (0, 0, 0)),
            pl.BlockSpec((2 * C, C), lambda b: (0, 0)),
            pl.BlockSpec((2, 1, C), lambda b: (0, 0, 0)),
        ],
        out_specs=pl.BlockSpec((1, T * H, N), lambda b: (b, 0, 0)),
        out_shape=jax.ShapeDtypeStruct((B, T * H, N), f32),
        compiler_params=pltpu.CompilerParams(
            vmem_limit_bytes=63 * 1024 * 1024),
    )(x, Wq, bq.reshape(T, 1, C), Wkv, bkv.reshape(2, 1, C))

    sc_topk = functools.partial(
        pl.kernel,
        out_type=jax.ShapeDtypeStruct((R, N), f32),
        mesh=plsc.VectorSubcoreMesh(core_axis_name="c", subcore_axis_name="s"),
        compiler_params=pltpu.CompilerParams(needs_layout_passes=False),
        scratch_types=[
            pltpu.VMEM((N,), f32),
            pltpu.VMEM((N,), jnp.int32),
            pltpu.VMEM((N,), f32),
            pltpu.VMEM((256,), jnp.int32),
        ],
    )(_sc_body)
    wfield = sc_topk(scores.reshape(R, N))

    out = pl.pallas_call(
        _body_b,
        grid=(B, T),
        in_specs=[
            pl.BlockSpec((1, N, C), lambda b, t: (b, 0, 0)),
            pl.BlockSpec((1, 1, H, N), lambda b, t: (b, t, 0, 0)),
            pl.BlockSpec((2 * C, C), lambda b, t: (0, 0)),
            pl.BlockSpec((2, 1, C), lambda b, t: (0, 0, 0)),
            pl.BlockSpec((1, C, C), lambda b, t: (t, 0, 0)),
        ],
        out_specs=pl.BlockSpec((1, N, C), lambda b, t: (b, 0, 0)),
        out_shape=jax.ShapeDtypeStruct((B, N, C), x.dtype),
        compiler_params=pltpu.CompilerParams(
            vmem_limit_bytes=63 * 1024 * 1024),
    )(x, wfield.reshape(B, T, H, N), Wkv, bkv.reshape(2, 1, C), We)
    return out
